# baseline (device time: 99360 ns/iter reference)
import jax
import jax.numpy as jnp
from jax import lax
from jax.experimental import pallas as pl
from jax.experimental.pallas import tpu as pltpu

N_DEV = 8
HQ_LOC = 8
DH = 128
SQ = 1024
D_MODEL = 1024
WINDOW = 128
SCALE = 0.08838834764831843
N_STAGES = 3


def kernel(x, Wq, K_ext, V_ext, Wo):
    def body(x_ref, wq_ref, k_hbm, v_hbm, wo_ref, out_ref,
             kv_vmem, cb, recv_buf, copy_sems, send_sems, recv_sems):
        p = lax.axis_index("i")

        barrier = pltpu.get_barrier_semaphore()
        partner = [p ^ 1, p ^ 3, p ^ 4]
        for q in partner:
            pl.semaphore_signal(
                barrier, inc=1,
                device_id=(q,), device_id_type=pl.DeviceIdType.MESH,
            )
        pl.semaphore_wait(barrier, N_STAGES)

        h0 = p * HQ_LOC
        k_copy = pltpu.make_async_copy(
            k_hbm.at[0, :, pl.ds(h0, HQ_LOC), :], kv_vmem.at[0],
            copy_sems.at[0])
        v_copy = pltpu.make_async_copy(
            v_hbm.at[0, :, pl.ds(h0, HQ_LOC), :], kv_vmem.at[1],
            copy_sems.at[1])
        k_copy.start()
        v_copy.start()

        xb = x_ref[0].astype(jnp.bfloat16)
        wq = wq_ref[...].astype(jnp.bfloat16)
        q_all = lax.dot_general(
            xb, wq, (((1,), (0,)), ((), ())),
            preferred_element_type=jnp.float32,
        ).astype(jnp.bfloat16).reshape(SQ, HQ_LOC, DH)

        k_copy.wait()
        v_copy.wait()
        k = kv_vmem[0].astype(jnp.bfloat16)
        v = kv_vmem[1].astype(jnp.bfloat16)

        wo = wo_ref[...].astype(jnp.bfloat16)

        QB, KB = 256, 512

        def compute_block(qs):
            ks = min(max(qs - WINDOW, 0), SQ - KB)
            qi = lax.broadcasted_iota(jnp.int32, (QB, KB), 0) + qs
            ki = lax.broadcasted_iota(jnp.int32, (QB, KB), 1) + ks
            mask = jnp.abs(qi - ki) <= WINDOW
            acc = jnp.zeros((QB, D_MODEL), jnp.float32)
            for h in range(HQ_LOC):
                qh = q_all[qs:qs + QB, h, :]
                kh = k[ks:ks + KB, h, :]
                sc = lax.dot_general(
                    qh, kh, (((1,), (1,)), ((), ())),
                    preferred_element_type=jnp.float32) * SCALE
                sc = jnp.where(mask, sc, -1e9)
                m = jnp.max(sc, axis=-1, keepdims=True)
                w = jnp.exp(sc - m)
                w = w / jnp.sum(w, axis=-1, keepdims=True)
                ctx_h = lax.dot_general(
                    w.astype(jnp.bfloat16), v[ks:ks + KB, h, :],
                    (((1,), (0,)), ((), ())),
                    preferred_element_type=jnp.float32)
                acc = acc + lax.dot_general(
                    ctx_h.astype(jnp.bfloat16), wo[h * DH:(h + 1) * DH, :],
                    (((1,), (0,)), ((), ())),
                    preferred_element_type=jnp.float32)
            cb[qs:qs + QB, :] = acc.astype(jnp.bfloat16)

        pp = p % 4
        cx = (pp ^ (pp >> 1)) & 1
        cy = pp >> 1
        cz = p // 4
        base1 = 512 * cx
        base2 = base1 + 256 * cy
        base3 = base2 + 128 * cz
        @pl.when(cx == 0)
        def _():
            compute_block(512)
            compute_block(768)

        @pl.when(cx == 1)
        def _():
            compute_block(0)
            compute_block(256)

        rdma0 = pltpu.make_async_remote_copy(
            src_ref=cb.at[pl.ds(512 * (1 - cx), 512), :],
            dst_ref=recv_buf.at[pl.ds(0, 512), :],
            send_sem=send_sems.at[0],
            recv_sem=recv_sems.at[0],
            device_id=(partner[0],),
            device_id_type=pl.DeviceIdType.MESH,
        )
        rdma0.start()

        @pl.when(cx == 0)
        def _():
            compute_block(0)
            compute_block(256)

        @pl.when(cx == 1)
        def _():
            compute_block(512)
            compute_block(768)

        rdma0.wait()
        cb[pl.ds(base1, 512), :] = (
            cb[pl.ds(base1, 512), :].astype(jnp.float32)
            + recv_buf[pl.ds(0, 512), :].astype(jnp.float32)
        ).astype(jnp.bfloat16)

        rs = [
            (partner[1], base2, base1 + 256 * (1 - cy), 256, 1),
            (partner[2], base3, base2 + 128 * (1 - cz), 128, 2),
        ]
        roff = 512
        for q, keep, send, rows, s in rs:
            rdma = pltpu.make_async_remote_copy(
                src_ref=cb.at[pl.ds(send, rows), :],
                dst_ref=recv_buf.at[pl.ds(roff, rows), :],
                send_sem=send_sems.at[s],
                recv_sem=recv_sems.at[s],
                device_id=(q,),
                device_id_type=pl.DeviceIdType.MESH,
            )
            rdma.start()
            rdma.wait()
            cb[pl.ds(keep, rows), :] = (
                cb[pl.ds(keep, rows), :].astype(jnp.float32)
                + recv_buf[pl.ds(roff, rows), :].astype(jnp.float32)
            ).astype(jnp.bfloat16)
            roff += rows

        ag = [
            (partner[2], base3, 128, 3),
            (partner[1], base2, 256, 4),
            (partner[0], base1, 512, 5),
        ]
        for q, start, rows, s in ag:
            rdma = pltpu.make_async_remote_copy(
                src_ref=cb.at[pl.ds(start, rows), :],
                dst_ref=cb.at[pl.ds(start, rows), :],
                send_sem=send_sems.at[s],
                recv_sem=recv_sems.at[s],
                device_id=(q,),
                device_id_type=pl.DeviceIdType.MESH,
            )
            rdma.start()
            rdma.wait()

        out_ref[0] = cb[...].astype(jnp.float32)

    return pl.pallas_call(
        body,
        out_shape=jax.ShapeDtypeStruct((1, SQ, D_MODEL), jnp.float32),
        in_specs=[
            pl.BlockSpec(memory_space=pltpu.VMEM),
            pl.BlockSpec(memory_space=pltpu.VMEM),
            pl.BlockSpec(memory_space=pl.ANY),
            pl.BlockSpec(memory_space=pl.ANY),
            pl.BlockSpec(memory_space=pltpu.VMEM),
        ],
        out_specs=pl.BlockSpec(memory_space=pltpu.VMEM),
        scratch_shapes=[
            pltpu.VMEM((2, SQ, HQ_LOC, DH), jnp.float32),
            pltpu.VMEM((SQ, D_MODEL), jnp.bfloat16),
            pltpu.VMEM((896, D_MODEL), jnp.bfloat16),
            pltpu.SemaphoreType.DMA((2,)),
            pltpu.SemaphoreType.DMA((6,)),
            pltpu.SemaphoreType.DMA((6,)),
        ],
        compiler_params=pltpu.CompilerParams(
            collective_id=0,
            vmem_limit_bytes=60 * 1024 * 1024,
        ),
    )(x, Wq, K_ext, V_ext, Wo)


# device time: 81001 ns/iter; 1.2267x vs baseline; 1.2267x over previous
import os

import jax
import jax.numpy as jnp
from jax import lax
from jax.experimental import pallas as pl
from jax.experimental.pallas import tpu as pltpu

_SKIP_COMM = os.environ.get("SKIP_COMM") == "1"

N_DEV = 8
HQ_LOC = 8
DH = 128
SQ = 1024
D_MODEL = 1024
WINDOW = 128
SCALE = 0.08838834764831843
N_STAGES = 3


def kernel(x, Wq, K_ext, V_ext, Wo):
    def body(x_ref, wq_ref, k_hbm, v_hbm, wo_ref, out_ref,
             kv_vmem, cb, recv_buf, copy_sems, send_sems, recv_sems):
        p = lax.axis_index("i")

        barrier = pltpu.get_barrier_semaphore()
        partner = [p ^ 1, p ^ 3, p ^ 4]
        for q in partner:
            pl.semaphore_signal(
                barrier, inc=1,
                device_id=(q,), device_id_type=pl.DeviceIdType.MESH,
            )
        pl.semaphore_wait(barrier, N_STAGES)

        h0 = p * HQ_LOC
        k_copy = pltpu.make_async_copy(
            k_hbm.at[0, :, pl.ds(h0, HQ_LOC), :], kv_vmem.at[0],
            copy_sems.at[0])
        v_copy = pltpu.make_async_copy(
            v_hbm.at[0, :, pl.ds(h0, HQ_LOC), :], kv_vmem.at[1],
            copy_sems.at[1])
        k_copy.start()
        v_copy.start()

        xb = x_ref[0].astype(jnp.bfloat16)
        wq = wq_ref[...].astype(jnp.bfloat16)
        q_all = (lax.dot_general(
            xb, wq, (((1,), (0,)), ((), ())),
            preferred_element_type=jnp.float32,
        ) * SCALE).astype(jnp.bfloat16).reshape(SQ, HQ_LOC, DH)

        k_copy.wait()
        v_copy.wait()
        k = kv_vmem[0].astype(jnp.bfloat16)
        v = kv_vmem[1].astype(jnp.bfloat16)

        wo = wo_ref[...].astype(jnp.bfloat16)

        QB, KB = 256, 512

        def compute_block(qs):
            ks = min(max(qs - WINDOW, 0), SQ - KB)
            qi = lax.broadcasted_iota(jnp.int32, (QB, KB), 0) + qs
            ki = lax.broadcasted_iota(jnp.int32, (QB, KB), 1) + ks
            bias = jnp.where(jnp.abs(qi - ki) <= WINDOW, 0.0, -1e9)
            acc = jnp.zeros((QB, D_MODEL), jnp.float32)
            for h in range(HQ_LOC):
                qh = q_all[qs:qs + QB, h, :]
                kh = k[ks:ks + KB, h, :]
                sc = lax.dot_general(
                    qh, kh, (((1,), (1,)), ((), ())),
                    preferred_element_type=jnp.float32)
                w = jnp.exp(sc + bias)
                recip = 1.0 / jnp.sum(w, axis=-1, keepdims=True)
                ctx_h = lax.dot_general(
                    w.astype(jnp.bfloat16), v[ks:ks + KB, h, :],
                    (((1,), (0,)), ((), ())),
                    preferred_element_type=jnp.float32) * recip
                acc = acc + lax.dot_general(
                    ctx_h.astype(jnp.bfloat16), wo[h * DH:(h + 1) * DH, :],
                    (((1,), (0,)), ((), ())),
                    preferred_element_type=jnp.float32)
            cb[qs:qs + QB, :] = acc.astype(jnp.bfloat16)

        pp = p % 4
        cx = (pp ^ (pp >> 1)) & 1
        cy = pp >> 1
        cz = p // 4
        base1 = 512 * cx
        base2 = base1 + 256 * cy
        base3 = base2 + 128 * cz
        @pl.when(cx == 0)
        def _():
            compute_block(512)
            compute_block(768)

        @pl.when(cx == 1)
        def _():
            compute_block(0)
            compute_block(256)

        if not _SKIP_COMM:
            rdma0 = pltpu.make_async_remote_copy(
                src_ref=cb.at[pl.ds(512 * (1 - cx), 512), :],
                dst_ref=recv_buf.at[pl.ds(0, 512), :],
                send_sem=send_sems.at[0],
                recv_sem=recv_sems.at[0],
                device_id=(partner[0],),
                device_id_type=pl.DeviceIdType.MESH,
            )
            rdma0.start()

        @pl.when(cx == 0)
        def _():
            compute_block(0)
            compute_block(256)

        @pl.when(cx == 1)
        def _():
            compute_block(512)
            compute_block(768)

        if _SKIP_COMM:
            out_ref[0] = cb[...].astype(jnp.float32)
            return

        rdma0.wait()
        cb[pl.ds(base1, 512), :] = (
            cb[pl.ds(base1, 512), :].astype(jnp.float32)
            + recv_buf[pl.ds(0, 512), :].astype(jnp.float32)
        ).astype(jnp.bfloat16)

        rs = [
            (partner[1], base2, base1 + 256 * (1 - cy), 256, 1),
            (partner[2], base3, base2 + 128 * (1 - cz), 128, 2),
        ]
        roff = 512
        for q, keep, send, rows, s in rs:
            rdma = pltpu.make_async_remote_copy(
                src_ref=cb.at[pl.ds(send, rows), :],
                dst_ref=recv_buf.at[pl.ds(roff, rows), :],
                send_sem=send_sems.at[s],
                recv_sem=recv_sems.at[s],
                device_id=(q,),
                device_id_type=pl.DeviceIdType.MESH,
            )
            rdma.start()
            rdma.wait()
            cb[pl.ds(keep, rows), :] = (
                cb[pl.ds(keep, rows), :].astype(jnp.float32)
                + recv_buf[pl.ds(roff, rows), :].astype(jnp.float32)
            ).astype(jnp.bfloat16)
            roff += rows

        ag = [
            (partner[2], base3, 128, 3),
            (partner[1], base2, 256, 4),
            (partner[0], base1, 512, 5),
        ]
        for q, start, rows, s in ag:
            rdma = pltpu.make_async_remote_copy(
                src_ref=cb.at[pl.ds(start, rows), :],
                dst_ref=cb.at[pl.ds(start, rows), :],
                send_sem=send_sems.at[s],
                recv_sem=recv_sems.at[s],
                device_id=(q,),
                device_id_type=pl.DeviceIdType.MESH,
            )
            rdma.start()
            rdma.wait()

        out_ref[0] = cb[...].astype(jnp.float32)

    return pl.pallas_call(
        body,
        out_shape=jax.ShapeDtypeStruct((1, SQ, D_MODEL), jnp.float32),
        in_specs=[
            pl.BlockSpec(memory_space=pltpu.VMEM),
            pl.BlockSpec(memory_space=pltpu.VMEM),
            pl.BlockSpec(memory_space=pl.ANY),
            pl.BlockSpec(memory_space=pl.ANY),
            pl.BlockSpec(memory_space=pltpu.VMEM),
        ],
        out_specs=pl.BlockSpec(memory_space=pltpu.VMEM),
        scratch_shapes=[
            pltpu.VMEM((2, SQ, HQ_LOC, DH), jnp.float32),
            pltpu.VMEM((SQ, D_MODEL), jnp.bfloat16),
            pltpu.VMEM((896, D_MODEL), jnp.bfloat16),
            pltpu.SemaphoreType.DMA((2,)),
            pltpu.SemaphoreType.DMA((6,)),
            pltpu.SemaphoreType.DMA((6,)),
        ],
        compiler_params=pltpu.CompilerParams(
            collective_id=0,
            vmem_limit_bytes=60 * 1024 * 1024,
        ),
    )(x, Wq, K_ext, V_ext, Wo)
